# baseline (speedup 1.0000x reference)
import functools

import jax
import jax.numpy as jnp
from jax import lax
from jax.experimental import pallas as pl
from jax.experimental.pallas import tpu as pltpu
from jax.experimental.pallas import tpu_sc as plsc

N = 10000
D = 128
DH = D // 2
E = 320000
NC = 2
NS = 16
C = 80
NCHUNK = E // (NS * C)
ZR = 400
NZCHUNK = N // ZR


def _sc_aggregate(yL, yR, b2, src, dst):
    mesh = plsc.VectorSubcoreMesh(core_axis_name="c", subcore_axis_name="s")

    @functools.partial(
        pl.kernel,
        out_type=jax.ShapeDtypeStruct((N, NC, DH), jnp.float32),
        mesh=mesh,
        compiler_params=pltpu.CompilerParams(use_tc_tiling_on_sc=False),
        scratch_types=[
            pltpu.VMEM((NCHUNK, C), jnp.int32),
            pltpu.VMEM((NCHUNK, C), jnp.int32),
            pltpu.VMEM((C, DH), jnp.float32),
            pltpu.VMEM((C, DH), jnp.float32),
            pltpu.VMEM((NC, DH), jnp.float32),
            pltpu.VMEM((ZR, DH), jnp.float32),
            pltpu.VMEM_SHARED((N, DH), jnp.float32),
            pltpu.SemaphoreType.DMA,
            pltpu.SemaphoreType.DMA,
        ],
    )
    def agg(yL_hbm, yR_hbm, b2_hbm, src_hbm, dst_hbm, out_hbm,
            src_v, dst_v, rows0_v, rows1_v, bias_v, zbuf, acc_sh, sem0, sem1):
        cid = lax.axis_index("c")
        sid = lax.axis_index("s")

        pltpu.sync_copy(src_hbm.at[sid], src_v)
        pltpu.sync_copy(dst_hbm.at[sid], dst_v)
        pltpu.sync_copy(b2_hbm, bias_v)

        def zbody(i, carry):
            r = i // (DH // 16)
            c0 = lax.rem(i, DH // 16)
            zbuf[r, pl.ds(c0 * 16, 16)] = bias_v[cid, pl.ds(c0 * 16, 16)]
            return carry
        lax.fori_loop(0, ZR * (DH // 16), zbody, 0)
        for k in range(NZCHUNK):
            @pl.when(sid == (k % NS))
            def _():
                pltpu.sync_copy(zbuf, acc_sh.at[pl.ds(k * ZR, ZR)])
        plsc.subcore_barrier()

        slots = ((rows0_v, sem0), (rows1_v, sem1))

        def start_gather(j, rows_v, sem):
            @pl.when(cid == 0)
            def _():
                pltpu.async_copy(yL_hbm.at[src_v.at[j]], rows_v, sem)

            @pl.when(cid == 1)
            def _():
                pltpu.async_copy(yR_hbm.at[src_v.at[j]], rows_v, sem)

        for b, (rows_v, sem) in enumerate(slots):
            start_gather(b, rows_v, sem)

        def body(g, carry):
            for b, (rows_v, sem) in enumerate(slots):
                j = g * 2 + b
                pltpu.make_async_copy(yL_hbm.at[src_v.at[j]], rows_v,
                                      sem).wait()
                pltpu.sync_copy(rows_v, acc_sh.at[dst_v.at[j]], add=True)

                @pl.when(j + 2 < NCHUNK)
                def _():
                    start_gather(j + 2, rows_v, sem)
            return carry
        lax.fori_loop(0, NCHUNK // 2, body, 0)
        plsc.subcore_barrier()

        for k in range(NZCHUNK):
            @pl.when(sid == (k % NS))
            def _():
                pltpu.sync_copy(acc_sh.at[pl.ds(k * ZR, ZR)],
                                out_hbm.at[pl.ds(k * ZR, ZR), cid])

    return agg(yL, yR, b2, src, dst)


def _tc_matmul(feature, W):
    BM = 1000

    def mm(f_ref, w_ref, oL_ref, oR_ref):
        y = jnp.dot(f_ref[...], w_ref[...], preferred_element_type=jnp.float32)
        oL_ref[...] = y[:, :DH]
        oR_ref[...] = y[:, DH:]

    return pl.pallas_call(
        mm,
        grid=(N // BM,),
        in_specs=[
            pl.BlockSpec((BM, D), lambda i: (i, 0)),
            pl.BlockSpec((D, D), lambda i: (0, 0)),
        ],
        out_specs=[
            pl.BlockSpec((BM, DH), lambda i: (i, 0)),
            pl.BlockSpec((BM, DH), lambda i: (i, 0)),
        ],
        out_shape=[
            jax.ShapeDtypeStruct((N, DH), jnp.float32),
            jax.ShapeDtypeStruct((N, DH), jnp.float32),
        ],
    )(feature, W)


def kernel(feature, edge_index, W, b):
    ei = edge_index.astype(jnp.int32)
    src = ei[0].reshape(NS, NCHUNK, C)
    dst = ei[1].reshape(NS, NCHUNK, C)
    yL, yR = _tc_matmul(feature, W)
    out3 = _sc_aggregate(yL, yR, b.reshape(NC, DH), src, dst)
    return out3.reshape(N, D)


# speedup vs baseline: 1.2493x; 1.2493x over previous
import functools

import jax
import jax.numpy as jnp
from jax import lax
from jax.experimental import pallas as pl
from jax.experimental.pallas import tpu as pltpu
from jax.experimental.pallas import tpu_sc as plsc

N = 10000
D = 128
DH = D // 2
E = 320000
NC = 2
NS = 16
C = 125
NCHUNK = E // (NS * C)
ZR = 400
NZCHUNK = N // ZR


def _sc_aggregate(featL, featR, src, dst):
    mesh = plsc.VectorSubcoreMesh(core_axis_name="c", subcore_axis_name="s")

    @functools.partial(
        pl.kernel,
        out_type=jax.ShapeDtypeStruct((NC, N, DH), jnp.float32),
        mesh=mesh,
        compiler_params=pltpu.CompilerParams(use_tc_tiling_on_sc=False),
        scratch_types=[
            pltpu.VMEM((NCHUNK, C), jnp.int32),
            pltpu.VMEM((NCHUNK, C), jnp.int32),
            pltpu.VMEM((C, DH), jnp.float32),
            pltpu.VMEM((C, DH), jnp.float32),
            pltpu.VMEM((ZR, DH), jnp.float32),
            pltpu.VMEM_SHARED((N, DH), jnp.float32),
            pltpu.SemaphoreType.DMA,
            pltpu.SemaphoreType.DMA,
        ],
    )
    def agg(featL_hbm, featR_hbm, src_hbm, dst_hbm, out_hbm,
            src_v, dst_v, rows0_v, rows1_v, zbuf, acc_sh, sem0, sem1):
        cid = lax.axis_index("c")
        sid = lax.axis_index("s")

        pltpu.sync_copy(src_hbm.at[sid], src_v)
        pltpu.sync_copy(dst_hbm.at[sid], dst_v)

        def zbody(i, carry):
            r = i // (DH // 16)
            c0 = lax.rem(i, DH // 16)
            zbuf[r, pl.ds(c0 * 16, 16)] = jnp.zeros((16,), jnp.float32)
            return carry
        lax.fori_loop(0, ZR * (DH // 16), zbody, 0)
        for k in range(NZCHUNK):
            @pl.when(sid == (k % NS))
            def _():
                pltpu.sync_copy(zbuf, acc_sh.at[pl.ds(k * ZR, ZR)])
        plsc.subcore_barrier()

        slots = ((rows0_v, sem0), (rows1_v, sem1))

        def start_gather(j, rows_v, sem):
            @pl.when(cid == 0)
            def _():
                pltpu.async_copy(featL_hbm.at[src_v.at[j]], rows_v, sem)

            @pl.when(cid == 1)
            def _():
                pltpu.async_copy(featR_hbm.at[src_v.at[j]], rows_v, sem)

        for b, (rows_v, sem) in enumerate(slots):
            start_gather(b, rows_v, sem)

        def body(g, carry):
            for b, (rows_v, sem) in enumerate(slots):
                j = g * 2 + b
                pltpu.make_async_copy(featL_hbm.at[src_v.at[j]], rows_v,
                                      sem).wait()
                pltpu.sync_copy(rows_v, acc_sh.at[dst_v.at[j]], add=True)

                @pl.when(j + 2 < NCHUNK)
                def _():
                    start_gather(j + 2, rows_v, sem)
            return carry
        lax.fori_loop(0, NCHUNK // 2, body, 0)
        plsc.subcore_barrier()

        for k in range(NZCHUNK):
            @pl.when(sid == (k % NS))
            def _():
                pltpu.sync_copy(acc_sh.at[pl.ds(k * ZR, ZR)],
                                out_hbm.at[cid, pl.ds(k * ZR, ZR)])

    return agg(featL, featR, src, dst)


def _tc_linear(partials, W, b2):
    BM = 1000

    def mm(p_ref, w_ref, b_ref, o_ref):
        h = jnp.concatenate([p_ref[0], p_ref[1]], axis=-1)
        o_ref[...] = (jnp.dot(h, w_ref[...], preferred_element_type=jnp.float32)
                      + b_ref[...])

    return pl.pallas_call(
        mm,
        grid=(N // BM,),
        in_specs=[
            pl.BlockSpec((NC, BM, DH), lambda i: (0, i, 0)),
            pl.BlockSpec((D, D), lambda i: (0, 0)),
            pl.BlockSpec((1, D), lambda i: (0, 0)),
        ],
        out_specs=pl.BlockSpec((BM, D), lambda i: (i, 0)),
        out_shape=jax.ShapeDtypeStruct((N, D), jnp.float32),
    )(partials, W, b2)


def kernel(feature, edge_index, W, b):
    ei = edge_index.astype(jnp.int32)
    src = ei[0].reshape(NS, NCHUNK, C)
    dst = ei[1].reshape(NS, NCHUNK, C)
    featL = feature[:, :DH]
    featR = feature[:, DH:]
    partials = _sc_aggregate(featL, featR, src, dst)
    return _tc_linear(partials, W, b.reshape(1, D))


# speedup vs baseline: 1.2674x; 1.0145x over previous
import functools

import jax
import jax.numpy as jnp
from jax import lax
from jax.experimental import pallas as pl
from jax.experimental.pallas import tpu as pltpu
from jax.experimental.pallas import tpu_sc as plsc

N = 10000
D = 128
DH = D // 2
E = 320000
NC = 2
NS = 16
C = 125
NCHUNK = E // (NS * C)
ZR = 80
NZCHUNK = N // ZR


def _sc_aggregate(featL, featR, src, dst):
    mesh = plsc.VectorSubcoreMesh(core_axis_name="c", subcore_axis_name="s")

    @functools.partial(
        pl.kernel,
        out_type=jax.ShapeDtypeStruct((NC, N, DH), jnp.float32),
        mesh=mesh,
        compiler_params=pltpu.CompilerParams(use_tc_tiling_on_sc=False),
        scratch_types=[
            pltpu.VMEM((NCHUNK, C), jnp.int32),
            pltpu.VMEM((NCHUNK, C), jnp.int32),
            pltpu.VMEM((C, DH), jnp.float32),
            pltpu.VMEM((C, DH), jnp.float32),
            pltpu.VMEM((ZR, DH), jnp.float32),
            pltpu.VMEM_SHARED((N, DH), jnp.float32),
            pltpu.SemaphoreType.DMA,
            pltpu.SemaphoreType.DMA,
        ],
    )
    def agg(featL_hbm, featR_hbm, src_hbm, dst_hbm, out_hbm,
            src_v, dst_v, rows0_v, rows1_v, zbuf, acc_sh, sem0, sem1):
        cid = lax.axis_index("c")
        sid = lax.axis_index("s")

        pltpu.sync_copy(src_hbm.at[sid], src_v)
        pltpu.sync_copy(dst_hbm.at[sid], dst_v)

        def start_gather(j, rows_v, sem):
            @pl.when(cid == 0)
            def _():
                pltpu.async_copy(featL_hbm.at[src_v.at[j]], rows_v, sem)

            @pl.when(cid == 1)
            def _():
                pltpu.async_copy(featR_hbm.at[src_v.at[j]], rows_v, sem)

        slots = ((rows0_v, sem0), (rows1_v, sem1))
        for b, (rows_v, sem) in enumerate(slots):
            start_gather(b, rows_v, sem)

        def zbody(i, carry):
            r = i // (DH // 16)
            c0 = lax.rem(i, DH // 16)
            zbuf[r, pl.ds(c0 * 16, 16)] = jnp.zeros((16,), jnp.float32)
            return carry
        lax.fori_loop(0, ZR * (DH // 16), zbody, 0)
        for k in range(NZCHUNK):
            @pl.when(sid == (k % NS))
            def _():
                pltpu.sync_copy(zbuf, acc_sh.at[pl.ds(k * ZR, ZR)])
        plsc.subcore_barrier()

        def body(g, carry):
            for b, (rows_v, sem) in enumerate(slots):
                j = g * 2 + b
                pltpu.make_async_copy(featL_hbm.at[src_v.at[j]], rows_v,
                                      sem).wait()
                pltpu.sync_copy(rows_v, acc_sh.at[dst_v.at[j]], add=True)

                @pl.when(j + 2 < NCHUNK)
                def _():
                    start_gather(j + 2, rows_v, sem)
            return carry
        lax.fori_loop(0, NCHUNK // 2, body, 0)
        plsc.subcore_barrier()

        for k in range(NZCHUNK):
            @pl.when(sid == (k % NS))
            def _():
                pltpu.sync_copy(acc_sh.at[pl.ds(k * ZR, ZR)],
                                out_hbm.at[cid, pl.ds(k * ZR, ZR)])

    return agg(featL, featR, src, dst)


def _tc_linear(partials, W, b2):
    BM = 1000

    def mm(p_ref, w_ref, b_ref, o_ref):
        h = jnp.concatenate([p_ref[0], p_ref[1]], axis=-1)
        o_ref[...] = (jnp.dot(h, w_ref[...], preferred_element_type=jnp.float32)
                      + b_ref[...])

    return pl.pallas_call(
        mm,
        grid=(N // BM,),
        in_specs=[
            pl.BlockSpec((NC, BM, DH), lambda i: (0, i, 0)),
            pl.BlockSpec((D, D), lambda i: (0, 0)),
            pl.BlockSpec((1, D), lambda i: (0, 0)),
        ],
        out_specs=pl.BlockSpec((BM, D), lambda i: (i, 0)),
        out_shape=jax.ShapeDtypeStruct((N, D), jnp.float32),
    )(partials, W, b2)


def kernel(feature, edge_index, W, b):
    ei = edge_index.astype(jnp.int32)
    src = ei[0].reshape(NS, NCHUNK, C)
    dst = ei[1].reshape(NS, NCHUNK, C)
    featL = feature[:, :DH]
    featR = feature[:, DH:]
    partials = _sc_aggregate(featL, featR, src, dst)
    return _tc_linear(partials, W, b.reshape(1, D))


# speedup vs baseline: 1.4720x; 1.1614x over previous
import functools

import jax
import jax.numpy as jnp
from jax import lax
from jax.experimental import pallas as pl
from jax.experimental.pallas import tpu as pltpu
from jax.experimental.pallas import tpu_sc as plsc

N = 10000
D = 128
DH = D // 2
E = 320000
NC = 2
NS = 16
C = 125
NCHUNK = E // (NS * C)
ZR = 80
NZCHUNK = N // ZR


def _sc_aggregate(featL, featR, src, dst):
    mesh = plsc.VectorSubcoreMesh(core_axis_name="c", subcore_axis_name="s")

    @functools.partial(
        pl.kernel,
        out_type=jax.ShapeDtypeStruct((NC, N, DH), jnp.bfloat16),
        mesh=mesh,
        compiler_params=pltpu.CompilerParams(use_tc_tiling_on_sc=False),
        scratch_types=[
            pltpu.VMEM((NCHUNK, C), jnp.int32),
            pltpu.VMEM((NCHUNK, C), jnp.int32),
            pltpu.VMEM((C, DH), jnp.bfloat16),
            pltpu.VMEM((C, DH), jnp.bfloat16),
            pltpu.VMEM((ZR, DH), jnp.bfloat16),
            pltpu.VMEM_SHARED((N, DH), jnp.bfloat16),
            pltpu.SemaphoreType.DMA,
            pltpu.SemaphoreType.DMA,
        ],
    )
    def agg(featL_hbm, featR_hbm, src_hbm, dst_hbm, out_hbm,
            src_v, dst_v, rows0_v, rows1_v, zbuf, acc_sh, sem0, sem1):
        cid = lax.axis_index("c")
        sid = lax.axis_index("s")

        pltpu.sync_copy(src_hbm.at[sid], src_v)
        pltpu.sync_copy(dst_hbm.at[sid], dst_v)

        def start_gather(j, rows_v, sem):
            @pl.when(cid == 0)
            def _():
                pltpu.async_copy(featL_hbm.at[src_v.at[j]], rows_v, sem)

            @pl.when(cid == 1)
            def _():
                pltpu.async_copy(featR_hbm.at[src_v.at[j]], rows_v, sem)

        slots = ((rows0_v, sem0), (rows1_v, sem1))
        for b, (rows_v, sem) in enumerate(slots):
            start_gather(b, rows_v, sem)

        def zbody(i, carry):
            r = i // (DH // 32)
            c0 = lax.rem(i, DH // 32)
            zbuf[r, pl.ds(c0 * 32, 32)] = jnp.zeros((32,), jnp.bfloat16)
            return carry
        lax.fori_loop(0, ZR * (DH // 32), zbody, 0)
        for k in range(NZCHUNK):
            @pl.when(sid == (k % NS))
            def _():
                pltpu.sync_copy(zbuf, acc_sh.at[pl.ds(k * ZR, ZR)])
        plsc.subcore_barrier()

        def body(g, carry):
            for b, (rows_v, sem) in enumerate(slots):
                j = g * 2 + b
                pltpu.make_async_copy(featL_hbm.at[src_v.at[j]], rows_v,
                                      sem).wait()
                pltpu.sync_copy(rows_v, acc_sh.at[dst_v.at[j]], add=True)

                @pl.when(j + 2 < NCHUNK)
                def _():
                    start_gather(j + 2, rows_v, sem)
            return carry
        lax.fori_loop(0, NCHUNK // 2, body, 0)
        plsc.subcore_barrier()

        for k in range(NZCHUNK):
            @pl.when(sid == (k % NS))
            def _():
                pltpu.sync_copy(acc_sh.at[pl.ds(k * ZR, ZR)],
                                out_hbm.at[cid, pl.ds(k * ZR, ZR)])

    return agg(featL, featR, src, dst)


def _tc_linear(partials, W, b2):
    BM = 1000

    def mm(p_ref, w_ref, b_ref, o_ref):
        h = jnp.concatenate([p_ref[0], p_ref[1]], axis=-1).astype(jnp.float32)
        o_ref[...] = (jnp.dot(h, w_ref[...], preferred_element_type=jnp.float32)
                      + b_ref[...])

    return pl.pallas_call(
        mm,
        grid=(N // BM,),
        in_specs=[
            pl.BlockSpec((NC, BM, DH), lambda i: (0, i, 0)),
            pl.BlockSpec((D, D), lambda i: (0, 0)),
            pl.BlockSpec((1, D), lambda i: (0, 0)),
        ],
        out_specs=pl.BlockSpec((BM, D), lambda i: (i, 0)),
        out_shape=jax.ShapeDtypeStruct((N, D), jnp.float32),
    )(partials, W, b2)


def kernel(feature, edge_index, W, b):
    ei = edge_index.astype(jnp.int32)
    src = ei[0].reshape(NS, NCHUNK, C)
    dst = ei[1].reshape(NS, NCHUNK, C)
    featL = feature[:, :DH].astype(jnp.bfloat16)
    featR = feature[:, DH:].astype(jnp.bfloat16)
    partials = _sc_aggregate(featL, featR, src, dst)
    return _tc_linear(partials, W, b.reshape(1, D))
